# baseline (speedup 1.0000x reference)
import functools

import numpy as np
import jax
import jax.numpy as jnp
from jax import lax
from jax.experimental import pallas as pl
from jax.experimental.pallas import tpu as pltpu
from jax.experimental.pallas import tpu_sc as plsc

_AUDIO_RATE = 22050.0 / 256.0
_SIZES = [x * _AUDIO_RATE for x in [2.23147392, 2.62519274, 3.74199546,
                                    5.78800454, 8.02371882]]
_LEVEL_N = [4096, 2048, 1024, 512, 256]
_LEVEL_STRIDE = [1.0, 2.0, 4.0, 8.0, 16.0]
_LEVEL_OFF = [0, 4096, 6144, 7168, 7680]
_LEVEL_LO = [0.0] + _SIZES[:4]
_LEVEL_UP = _SIZES

_B, _G, _C = 16, 30, 8
_A = sum(_LEVEL_N)
_ROWS = _A // 128
_WS = 4096
_NC, _NS = 2, 16


def _i_ge(q):
    qc = jnp.maximum(q, 0.0)
    t = qc.astype(jnp.int32)
    return t + (t.astype(jnp.float32) < qc).astype(jnp.int32)


def _i_gt(q):
    t = jnp.maximum(q, 0.0).astype(jnp.int32)
    return jnp.where(q < 0.0, 0, t + 1)


def _i_le(q):
    t = jnp.maximum(q, 0.0).astype(jnp.int32)
    return jnp.where(q < 0.0, -1, t)


def _i_lt(q):
    qc = jnp.maximum(q, 0.0)
    t = qc.astype(jnp.int32)
    c = t + (t.astype(jnp.float32) < qc).astype(jnp.int32)
    return jnp.where(q <= 0.0, -1, c - 1)


def _sc_mask_body(ann_hbm, cid_hbm, out_hbm, ann_v, cid_v, mask_v):
    b = lax.axis_index("s")
    h = lax.axis_index("c")
    wbase = h * _WS

    lanes = lax.iota(jnp.int32, 16)
    zeros16 = jnp.zeros((16,), jnp.float32)

    def zero_body(i, _):
        mask_v[i >> 3, pl.ds((i & 7) * 16, 16)] = zeros16
        return 0
    lax.fori_loop(0, _WS // 16, zero_body, 0)

    pltpu.sync_copy(ann_hbm.at[b], ann_v)
    pltpu.sync_copy(cid_hbm, cid_v)
    cidf = cid_v[...][0].astype(jnp.float32)

    def fill(glo, ghi, xlo, xhi, nchunks):
        llo = jnp.maximum(glo, wbase) - wbase
        lhi = jnp.minimum(ghi, wbase + _WS - 1) - wbase
        exlo = xlo - wbase
        exhi = xhi - wbase
        cstart = jnp.clip((llo // 16) * 16, 0, _WS - 16)
        for i in range(nchunks):
            c0 = jnp.minimum(cstart + 16 * i, _WS - 16)
            idx = lanes + c0
            m = ((idx >= llo) & (idx <= lhi)
                 & ((idx < exlo) | (idx > exhi)))
            row = c0 >> 7
            col = c0 & 127
            v = mask_v[row, pl.ds(col, 16)]
            mask_v[row, pl.ds(col, 16)] = jnp.where(m, 1.0, v)

    def g_body(g, _):
        av = ann_v[pl.ds(4 * g, 16)]
        s = av[0]
        e = av[1]
        cl = av[2]

        @pl.when(cl == cidf)
        def _():
            chunks = [14, 10, 8, 6, 5]
            for lvl in range(5):
                inv = 1.0 / _LEVEL_STRIDE[lvl]
                off = _LEVEL_OFF[lvl]
                n = _LEVEL_N[lvl]
                lo = _LEVEL_LO[lvl]
                up = _LEVEL_UP[lvl]
                lk = jnp.maximum(_i_ge(s * inv), _i_gt((e - up) * inv))
                hk = jnp.minimum(_i_le(e * inv), _i_lt((s + up) * inv))
                lk = jnp.maximum(lk, 0)
                hk = jnp.minimum(hk, n - 1)
                xl = _i_gt((e - lo) * inv)
                xh = _i_lt((s + lo) * inv)
                fill(lk + off, hk + off, xl + off, xh + off, chunks[lvl])
        return 0

    lax.fori_loop(0, _G, g_body, 0)

    pltpu.sync_copy(mask_v, out_hbm.at[b, pl.ds(h * 32, 32)])


def _sc_mask(ann4, cid_arr):
    mesh = plsc.VectorSubcoreMesh(core_axis_name="c", subcore_axis_name="s",
                                  num_cores=_NC, num_subcores=_NS)
    return pl.kernel(
        _sc_mask_body,
        out_type=jax.ShapeDtypeStruct((_B, 64, 128), jnp.float32),
        mesh=mesh,
        scratch_types=[
            pltpu.VMEM((144,), jnp.float32),
            pltpu.VMEM((16,), jnp.int32),
            pltpu.VMEM((32, 128), jnp.float32),
        ],
    )(ann4, cid_arr)


def _tca_kernel(cid_ref, x_hbm, negs_ref, delta_ref, x_ref, dma_sem):
    cid = cid_ref[0, 0]

    copy = pltpu.make_async_copy(x_hbm, x_ref, dma_sem)
    copy.start()
    copy.wait()

    def b_body(b, carry):
        x = x_ref[b]
        cls = jnp.clip(x, 1e-4, 1.0 - 1e-4)
        neg = 0.75 * cls * cls * (-jnp.log(1.0 - cls))
        negs_ref[b, 0] = jnp.sum(neg)

        xc = x_ref[b, pl.ds(cid * _ROWS, _ROWS), :]
        cc = jnp.clip(xc, 1e-4, 1.0 - 1e-4)
        one_m = 1.0 - cc
        post = 0.25 * one_m * one_m * (-jnp.log(cc))
        negt = 0.75 * cc * cc * (-jnp.log(one_m))
        delta_ref[b] = post - negt
        return carry

    lax.fori_loop(0, _B, b_body, 0)


def _tcb_kernel(negs_ref, m_ref, d_ref, out_ref):
    out_ref[0, 0] = 0.0

    def b_body(b, carry):
        m = m_ref[b, 0:_ROWS, :]
        npos = jnp.sum(m)
        corr = jnp.sum(m * d_ref[b])
        out_ref[0, 0] += ((negs_ref[b, 0] + corr)
                          / jnp.maximum(npos, 1.0)) / _B
        return carry

    lax.fori_loop(0, _B, b_body, 0)


def kernel(classifications, annotations, anchors0, anchors1, anchors2,
           anchors3, anchors4, class_id):
    B, A, C = classifications.shape
    ann4 = jnp.pad(
        jnp.pad(annotations, ((0, 0), (0, 2), (0, 1))).reshape(B, 128),
        ((0, 0), (0, 16)))
    cid_arr = jnp.full((16,), class_id, jnp.int32)
    cid = jnp.asarray(class_id, jnp.int32).reshape(1, 1)
    xt = jnp.transpose(classifications, (0, 2, 1)).reshape(B, C * _ROWS, 128)

    mask = _sc_mask(ann4, cid_arr)

    negs, delta = pl.pallas_call(
        _tca_kernel,
        in_specs=[
            pl.BlockSpec(memory_space=pltpu.SMEM),
            pl.BlockSpec(memory_space=pl.ANY),
        ],
        out_specs=[
            pl.BlockSpec(memory_space=pltpu.SMEM),
            pl.BlockSpec(memory_space=pltpu.VMEM),
        ],
        out_shape=[
            jax.ShapeDtypeStruct((_B, 1), jnp.float32),
            jax.ShapeDtypeStruct((_B, _ROWS, 128), jnp.float32),
        ],
        scratch_shapes=[
            pltpu.VMEM((_B, _C * _ROWS, 128), jnp.float32),
            pltpu.SemaphoreType.DMA,
        ],
    )(cid, xt)

    out = pl.pallas_call(
        _tcb_kernel,
        in_specs=[
            pl.BlockSpec(memory_space=pltpu.SMEM),
            pl.BlockSpec(memory_space=pltpu.VMEM),
            pl.BlockSpec(memory_space=pltpu.VMEM),
        ],
        out_specs=pl.BlockSpec(memory_space=pltpu.SMEM),
        out_shape=jax.ShapeDtypeStruct((1, 1), jnp.float32),
    )(negs, mask, delta)
    return out[0, 0]


# speedup vs baseline: 1.3122x; 1.3122x over previous
import numpy as np
import jax
import jax.numpy as jnp
from jax import lax
from jax.experimental import pallas as pl
from jax.experimental.pallas import tpu as pltpu

_AUDIO_RATE = 22050.0 / 256.0
_SIZES = [x * _AUDIO_RATE for x in [2.23147392, 2.62519274, 3.74199546,
                                    5.78800454, 8.02371882]]

_B, _G, _C = 16, 30, 8
_A = 4096 + 2048 + 1024 + 512 + 256
_ROWS = _A // 128


def _focal_kernel(ann_ref, cid_ref, x_hbm, out_ref, x_ref, pos_ref,
                  dma_sem):
    cid = cid_ref[0, 0]
    cidf = cid.astype(jnp.float32)

    copy = pltpu.make_async_copy(x_hbm, x_ref, dma_sem)
    copy.start()

    ri = lax.broadcasted_iota(jnp.int32, (_ROWS, 128), 0)
    ci = lax.broadcasted_iota(jnp.int32, (_ROWS, 128), 1)
    af = (ri * 128 + ci).astype(jnp.float32)
    s0, s1, s2, s3 = _SIZES[0], _SIZES[1], _SIZES[2], _SIZES[3]
    p = jnp.where(
        af < 4096.0, af,
        jnp.where(af < 6144.0, 2.0 * (af - 4096.0),
                  jnp.where(af < 7168.0, 4.0 * (af - 6144.0),
                            jnp.where(af < 7680.0, 8.0 * (af - 7168.0),
                                      16.0 * (af - 7680.0)))))
    lo = jnp.where(
        af < 4096.0, 0.0,
        jnp.where(af < 6144.0, s0,
                  jnp.where(af < 7168.0, s1,
                            jnp.where(af < 7680.0, s2, s3))))
    up = jnp.where(
        af < 4096.0, _SIZES[0],
        jnp.where(af < 6144.0, _SIZES[1],
                  jnp.where(af < 7168.0, _SIZES[2],
                            jnp.where(af < 7680.0, _SIZES[3], _SIZES[4]))))

    pos_ref[...] = jnp.zeros((_B, _ROWS, 128), jnp.float32)

    for b in range(_B):
        def g_body(g, carry, b=b):
            cl = ann_ref[b, g, 2]

            @pl.when(cl == cidf)
            def _():
                s = ann_ref[b, g, 0]
                e = ann_ref[b, g, 1]
                l = p - s
                r = e - p
                mn = jnp.minimum(l, r)
                mx = jnp.maximum(l, r)
                q = jnp.minimum(mn, mx - lo)
                ok = (q >= 0.0) & (mx < up)
                pos_ref[b] = jnp.maximum(pos_ref[b],
                                         jnp.where(ok, 1.0, 0.0))
            return carry

        lax.fori_loop(0, _G, g_body, 0)

    copy.wait()
    out_ref[0, 0] = 0.0

    def b_body(b, carry):
        x = x_ref[b]
        cls = jnp.clip(x, 1e-4, 1.0 - 1e-4)
        neg = 0.75 * cls * cls * (-jnp.log(1.0 - cls))
        negs = jnp.sum(neg)

        posf = pos_ref[b]
        npos = jnp.sum(posf)

        xc = x_ref[b, pl.ds(cid * _ROWS, _ROWS), :]
        cc = jnp.clip(xc, 1e-4, 1.0 - 1e-4)
        one_m = 1.0 - cc
        post = 0.25 * one_m * one_m * (-jnp.log(cc))
        negt = 0.75 * cc * cc * (-jnp.log(one_m))
        corr = jnp.sum(posf * (post - negt))

        out_ref[0, 0] += ((negs + corr)
                          / jnp.maximum(npos, 1.0)) / _B
        return carry

    lax.fori_loop(0, _B, b_body, 0)


def kernel(classifications, annotations, anchors0, anchors1, anchors2,
           anchors3, anchors4, class_id):
    B, A, C = classifications.shape
    xt = jnp.transpose(classifications, (0, 2, 1)).reshape(B, C * _ROWS, 128)
    cid = jnp.asarray(class_id, jnp.int32).reshape(1, 1)

    out = pl.pallas_call(
        _focal_kernel,
        in_specs=[
            pl.BlockSpec(memory_space=pltpu.SMEM),
            pl.BlockSpec(memory_space=pltpu.SMEM),
            pl.BlockSpec(memory_space=pl.ANY),
        ],
        out_specs=pl.BlockSpec(memory_space=pltpu.SMEM),
        out_shape=jax.ShapeDtypeStruct((1, 1), jnp.float32),
        scratch_shapes=[
            pltpu.VMEM((_B, _C * _ROWS, 128), jnp.float32),
            pltpu.VMEM((_B, _ROWS, 128), jnp.float32),
            pltpu.SemaphoreType.DMA,
        ],
    )(annotations, cid, xt)
    return out[0, 0]


# speedup vs baseline: 1.4649x; 1.1163x over previous
import numpy as np
import jax
import jax.numpy as jnp
from jax import lax
from jax.experimental import pallas as pl
from jax.experimental.pallas import tpu as pltpu

_AUDIO_RATE = 22050.0 / 256.0
_SIZES = [x * _AUDIO_RATE for x in [2.23147392, 2.62519274, 3.74199546,
                                    5.78800454, 8.02371882]]

_B, _G, _C = 16, 30, 8
_A = 4096 + 2048 + 1024 + 512 + 256
_ROWS = _A // 128


def _focal_kernel(ann_ref, cid_ref, x_hbm, out_ref, x_ref, pos_ref,
                  dma_sem):
    cid = cid_ref[0, 0]
    cidf = cid.astype(jnp.float32)

    copy = pltpu.make_async_copy(x_hbm, x_ref, dma_sem)
    copy.start()

    ri = lax.broadcasted_iota(jnp.int32, (_ROWS, 128), 0)
    ci = lax.broadcasted_iota(jnp.int32, (_ROWS, 128), 1)
    af = (ri * 128 + ci).astype(jnp.float32)
    s0, s1, s2, s3 = _SIZES[0], _SIZES[1], _SIZES[2], _SIZES[3]
    p = jnp.where(
        af < 4096.0, af,
        jnp.where(af < 6144.0, 2.0 * (af - 4096.0),
                  jnp.where(af < 7168.0, 4.0 * (af - 6144.0),
                            jnp.where(af < 7680.0, 8.0 * (af - 7168.0),
                                      16.0 * (af - 7680.0)))))
    lo = jnp.where(
        af < 4096.0, 0.0,
        jnp.where(af < 6144.0, s0,
                  jnp.where(af < 7168.0, s1,
                            jnp.where(af < 7680.0, s2, s3))))
    up = jnp.where(
        af < 4096.0, _SIZES[0],
        jnp.where(af < 6144.0, _SIZES[1],
                  jnp.where(af < 7168.0, _SIZES[2],
                            jnp.where(af < 7680.0, _SIZES[3], _SIZES[4]))))

    pos_ref[...] = jnp.zeros((_B, _ROWS, 128), jnp.float32)

    for b in range(_B):
        def g_body(g, carry, b=b):
            cl = ann_ref[b, g, 2]

            @pl.when(cl == cidf)
            def _():
                s = ann_ref[b, g, 0]
                e = ann_ref[b, g, 1]
                l = p - s
                r = e - p
                mn = jnp.minimum(l, r)
                mx = jnp.maximum(l, r)
                q = jnp.minimum(mn, mx - lo)
                ok = (q >= 0.0) & (mx < up)
                pos_ref[b] = jnp.maximum(pos_ref[b],
                                         jnp.where(ok, 1.0, 0.0))
            return carry

        lax.fori_loop(0, _G, g_body, 0)

    copy.wait()
    acc = 0.0
    for b in range(_B):
        x = x_ref[b]
        cls = jnp.minimum(x, 1.0 - 1e-4)
        neg = 0.75 * cls * cls * (-jnp.log(1.0 - cls))
        negs = jnp.sum(neg)

        posf = pos_ref[b]
        npos = jnp.sum(posf)

        xc = x_ref[b, pl.ds(cid * _ROWS, _ROWS), :]
        cc = jnp.clip(xc, 1e-4, 1.0 - 1e-4)
        one_m = 1.0 - cc
        post = 0.25 * one_m * one_m * (-jnp.log(cc))
        negt = 0.75 * cc * cc * (-jnp.log(one_m))
        corr = jnp.sum(posf * (post - negt))

        acc += ((negs + corr) / jnp.maximum(npos, 1.0)) / _B

    out_ref[0, 0] = acc


def kernel(classifications, annotations, anchors0, anchors1, anchors2,
           anchors3, anchors4, class_id):
    B, A, C = classifications.shape
    xt = jnp.transpose(classifications, (0, 2, 1)).reshape(B, C * _ROWS, 128)
    cid = jnp.asarray(class_id, jnp.int32).reshape(1, 1)

    out = pl.pallas_call(
        _focal_kernel,
        in_specs=[
            pl.BlockSpec(memory_space=pltpu.SMEM),
            pl.BlockSpec(memory_space=pltpu.SMEM),
            pl.BlockSpec(memory_space=pl.ANY),
        ],
        out_specs=pl.BlockSpec(memory_space=pltpu.SMEM),
        out_shape=jax.ShapeDtypeStruct((1, 1), jnp.float32),
        scratch_shapes=[
            pltpu.VMEM((_B, _C * _ROWS, 128), jnp.float32),
            pltpu.VMEM((_B, _ROWS, 128), jnp.float32),
            pltpu.SemaphoreType.DMA,
        ],
    )(annotations, cid, xt)
    return out[0, 0]
